# baseline (speedup 1.0000x reference)
import functools

import jax
import jax.numpy as jnp
from jax import lax
from jax.experimental import pallas as pl
from jax.experimental.pallas import tpu as pltpu
from jax.experimental.pallas import tpu_sc as plsc

B = 8
N = 8192
S = 1024
NROW = 64
NCOL = 128
SROW = 8
CIN = 128
COUT = 256


G = 1
GB = B // G


def _fps_body(x0_ref, x1_ref, x2_ref, xall_ref, idx_ref, n0_ref, n1_ref,
              n2_ref, *scratch):
    colv = lax.broadcasted_iota(jnp.int32, (1, NCOL), 1)
    rowio = lax.broadcasted_iota(jnp.int32, (GB, NROW, NCOL), 1)
    ones_mat = jnp.ones((NCOL, NCOL), jnp.float32)
    dists = scratch[0:G]
    for g in range(G):
        dists[g][...] = jnp.full((GB, NROW, NCOL), 1e10, jnp.float32)

    def s1(g, j, fx_g, acc):
        far_g = tuple(fx_g[bl, 0] for bl in range(GB))
        b0 = g * GB
        chunk = j // NCOL
        col = j - chunk * NCOL
        cmaskg = jnp.broadcast_to(colv == col, (GB, NCOL))
        base_row = (lax.broadcasted_iota(jnp.int32, (GB, NCOL), 0) + b0) * N
        ai, a0, a1, a2 = acc

        e0, e1, e2 = [], [], []
        for bl in range(GB):
            f = far_g[bl]
            r = f // NCOL
            c = f - r * NCOL
            lmask = colv == c
            rows = xall_ref[b0 + bl, pl.ds(r, 1)].reshape(3, NCOL)
            e0.append(jnp.where(lmask, rows[0:1, :], 0.0))
            e1.append(jnp.where(lmask, rows[1:2, :], 0.0))
            e2.append(jnp.where(lmask, rows[2:3, :], 0.0))

        def onehot_dot(es):
            return lax.dot_general(jnp.concatenate(es, axis=0), ones_mat,
                                   (((1,), (0,)), ((), ())),
                                   preferred_element_type=jnp.float32,
                                   precision=lax.Precision.HIGHEST)
        C0 = onehot_dot(e0)
        C1 = onehot_dot(e1)
        C2 = onehot_dot(e2)

        ai = jnp.where(cmaskg, jnp.broadcast_to(fx_g, (GB, NCOL))
                       + base_row, ai)
        a0 = jnp.where(cmaskg, C0, a0)
        a1 = jnp.where(cmaskg, C1, a1)
        a2 = jnp.where(cmaskg, C2, a2)
        new_acc = (ai, a0, a1, a2)

        def store_acc():
            for bl in range(GB):
                idx_ref[b0 + bl, pl.ds(chunk, 1), :] = ai[bl:bl + 1, :]
                n0_ref[b0 + bl, pl.ds(chunk, 1), :] = a0[bl:bl + 1, :]
                n1_ref[b0 + bl, pl.ds(chunk, 1), :] = a1[bl:bl + 1, :]
                n2_ref[b0 + bl, pl.ds(chunk, 1), :] = a2[bl:bl + 1, :]

        shp = (GB, NROW, NCOL)
        d0 = x0_ref[b0:b0 + GB] - jnp.broadcast_to(C0[:, None, :], shp)
        d1 = x1_ref[b0:b0 + GB] - jnp.broadcast_to(C1[:, None, :], shp)
        d2 = x2_ref[b0:b0 + GB] - jnp.broadcast_to(C2[:, None, :], shp)
        d = (d0 * d0 + d1 * d1) + d2 * d2
        dmin = jnp.minimum(dists[g][...], d)
        dists[g][...] = dmin

        m8 = jnp.max(dmin, axis=1)
        m8b = jnp.broadcast_to(m8[:, None, :], shp)
        selr = jnp.where(dmin == m8b, rowio, jnp.int32(2 ** 30))
        rmin8 = jnp.min(selr, axis=1)
        return m8, rmin8, new_acc, store_acc

    def s2(m8, rmin8):
        mx = jnp.max(m8, axis=1, keepdims=True)
        candf = (rmin8 * NCOL +
                 lax.broadcasted_iota(jnp.int32, (GB, NCOL), 1)
                 ).astype(jnp.float32)
        return jnp.where(m8 == jnp.broadcast_to(mx, (GB, NCOL)), candf,
                         jnp.float32(2 ** 30))

    def s3(candf):
        return jnp.min(candf, axis=1, keepdims=True).astype(jnp.int32)

    fx0 = jnp.zeros((GB, 1), jnp.int32)
    zacc = (jnp.zeros((GB, NCOL), jnp.int32),
            jnp.zeros((GB, NCOL), jnp.float32),
            jnp.zeros((GB, NCOL), jnp.float32),
            jnp.zeros((GB, NCOL), jnp.float32))
    m8a0, rmin8a0, acc_a, st0 = s1(0, 0, fx0, zacc)
    st0()
    fx_a = s3(s2(m8a0, rmin8a0))

    def body(k, carry):
        fx_a, acc_a = carry
        m8a, rmin8a, acc_a, store_acc = s1(0, k + 1, fx_a, acc_a)
        store_acc()
        cand_a = s2(m8a, rmin8a)
        new_fx_a = s3(cand_a)
        return new_fx_a, acc_a

    lax.fori_loop(0, S - 1, body, (fx_a, acc_a), unroll=False)


def _fps_pallas(x0, x1, x2, interpret=False):
    xall = jnp.stack([x0, x1, x2], axis=2)
    out_shape = [
        jax.ShapeDtypeStruct((B, SROW, NCOL), jnp.int32),
        jax.ShapeDtypeStruct((B, SROW, NCOL), jnp.float32),
        jax.ShapeDtypeStruct((B, SROW, NCOL), jnp.float32),
        jax.ShapeDtypeStruct((B, SROW, NCOL), jnp.float32),
    ]
    scratch = []
    for g in range(G):
        scratch.append(pltpu.VMEM((GB, NROW, NCOL), jnp.float32))
    return pl.pallas_call(
        _fps_body,
        out_shape=out_shape,
        scratch_shapes=scratch,
        interpret=interpret,
    )(x0, x1, x2, xall)


def _mm_body(g_ref, w_ref, bias_ref, out_ref):
    for b in range(B):
        out_ref[b] = lax.dot_general(
            w_ref[...], g_ref[b], (((1,), (1,)), ((), ())),
            preferred_element_type=jnp.float32,
            precision=lax.Precision.HIGHEST) + bias_ref[...]


def _mm_pallas(g, w, bias, interpret=False):
    return pl.pallas_call(
        _mm_body,
        out_shape=jax.ShapeDtypeStruct((B, COUT, S), jnp.float32),
        interpret=interpret,
    )(g, w, bias)


_BPW = (B * S) // 32
_IDX_ROWS = _BPW // NCOL


def _sc_gather_body(table_ref, idx_ref, out_ref, idx_v, rows_v, sem):
    wid = lax.axis_index("s") * 2 + lax.axis_index("c")
    pltpu.sync_copy(idx_ref.at[pl.ds(wid * _IDX_ROWS, _IDX_ROWS)], idx_v)
    for j in range(_IDX_ROWS):
        pltpu.async_copy(table_ref.at[idx_v.at[j]],
                         rows_v.at[pl.ds(j * NCOL, NCOL)], sem).wait()
    pltpu.sync_copy(rows_v, out_ref.at[pl.ds(wid * _BPW, _BPW)])


def _sc_gather(table, idx2d):
    mesh = plsc.VectorSubcoreMesh(core_axis_name="c", subcore_axis_name="s")
    kern = pl.kernel(
        _sc_gather_body,
        mesh=mesh,
        out_type=jax.ShapeDtypeStruct((B * S, CIN), jnp.float32),
        scratch_types=[
            pltpu.VMEM((_IDX_ROWS, NCOL), jnp.int32),
            pltpu.VMEM((_BPW, CIN), jnp.float32),
            pltpu.SemaphoreType.DMA,
        ],
    )
    return kern(table, idx2d)


def kernel(xyz, x, W, b):
    x0 = xyz[:, :, 0].reshape(B, NROW, NCOL)
    x1 = xyz[:, :, 1].reshape(B, NROW, NCOL)
    x2 = xyz[:, :, 2].reshape(B, NROW, NCOL)
    idx, n0, n1, n2 = _fps_pallas(x0, x1, x2)
    new_xyz = jnp.stack([n0.reshape(B, S), n1.reshape(B, S),
                         n2.reshape(B, S)], axis=-1)
    table = jnp.transpose(x, (0, 2, 1)).reshape(B * N, CIN)
    g = _sc_gather(table, idx.reshape((B * S) // NCOL, NCOL))
    new_x = _mm_pallas(g.reshape(B, S, CIN), W, b.reshape(COUT, 1))
    return (new_xyz, new_x)


# speedup vs baseline: 1.0133x; 1.0133x over previous
import jax
import jax.numpy as jnp
from jax import lax
from jax.experimental import pallas as pl
from jax.experimental.pallas import tpu as pltpu
from jax.experimental.pallas import tpu_sc as plsc

B = 8
N = 8192
S = 1024
NROW = 64
NCOL = 128
SROW = 8
CIN = 128
COUT = 256


G = 1
GB = B // G


def _fps_body(x0_ref, x1_ref, x2_ref, idx_ref, n0_ref, n1_ref,
              n2_ref, *scratch):
    colv = lax.broadcasted_iota(jnp.int32, (1, NCOL), 1)
    rowio = lax.broadcasted_iota(jnp.int32, (GB, NROW, NCOL), 1)
    ones_mat = jnp.ones((NCOL, NCOL), jnp.float32)
    dists = scratch[0:G]
    for g in range(G):
        dists[g][...] = jnp.full((GB, NROW, NCOL), 1e10, jnp.float32)

    def s1(g, j, fx_g, acc):
        far_g = tuple(fx_g[bl, 0] for bl in range(GB))
        b0 = g * GB
        chunk = j // NCOL
        col = j - chunk * NCOL
        cmaskg = jnp.broadcast_to(colv == col, (GB, NCOL))
        base_row = (lax.broadcasted_iota(jnp.int32, (GB, NCOL), 0) + b0) * N
        ai, a0, a1, a2 = acc

        e0, e1, e2 = [], [], []
        for bl in range(GB):
            f = far_g[bl]
            r = f // NCOL
            c = f - r * NCOL
            lmask = colv == c
            e0.append(jnp.where(lmask, x0_ref[b0 + bl, pl.ds(r, 1), :], 0.0))
            e1.append(jnp.where(lmask, x1_ref[b0 + bl, pl.ds(r, 1), :], 0.0))
            e2.append(jnp.where(lmask, x2_ref[b0 + bl, pl.ds(r, 1), :], 0.0))

        def onehot_dot(es):
            return lax.dot_general(jnp.concatenate(es, axis=0), ones_mat,
                                   (((1,), (0,)), ((), ())),
                                   preferred_element_type=jnp.float32,
                                   precision=lax.Precision.HIGHEST)
        C0 = onehot_dot(e0)
        C1 = onehot_dot(e1)
        C2 = onehot_dot(e2)

        ai = jnp.where(cmaskg, jnp.broadcast_to(fx_g, (GB, NCOL))
                       + base_row, ai)
        a0 = jnp.where(cmaskg, C0, a0)
        a1 = jnp.where(cmaskg, C1, a1)
        a2 = jnp.where(cmaskg, C2, a2)
        new_acc = (ai, a0, a1, a2)

        def store_acc():
            for bl in range(GB):
                idx_ref[b0 + bl, pl.ds(chunk, 1), :] = ai[bl:bl + 1, :]
                n0_ref[b0 + bl, pl.ds(chunk, 1), :] = a0[bl:bl + 1, :]
                n1_ref[b0 + bl, pl.ds(chunk, 1), :] = a1[bl:bl + 1, :]
                n2_ref[b0 + bl, pl.ds(chunk, 1), :] = a2[bl:bl + 1, :]

        shp = (GB, NROW, NCOL)
        d0 = x0_ref[b0:b0 + GB] - jnp.broadcast_to(C0[:, None, :], shp)
        d1 = x1_ref[b0:b0 + GB] - jnp.broadcast_to(C1[:, None, :], shp)
        d2 = x2_ref[b0:b0 + GB] - jnp.broadcast_to(C2[:, None, :], shp)
        d = (d0 * d0 + d1 * d1) + d2 * d2
        dmin = jnp.minimum(dists[g][...], d)
        dists[g][...] = dmin

        m8 = jnp.max(dmin, axis=1)
        m8b = jnp.broadcast_to(m8[:, None, :], shp)
        selr = jnp.where(dmin == m8b, rowio, jnp.int32(2 ** 30))
        rmin8 = jnp.min(selr, axis=1)
        return m8, rmin8, new_acc, store_acc

    def s2(m8, rmin8):
        mx = jnp.max(m8, axis=1, keepdims=True)
        candf = (rmin8 * NCOL +
                 lax.broadcasted_iota(jnp.int32, (GB, NCOL), 1)
                 ).astype(jnp.float32)
        return jnp.where(m8 == jnp.broadcast_to(mx, (GB, NCOL)), candf,
                         jnp.float32(2 ** 30))

    def s3(candf):
        return jnp.min(candf, axis=1, keepdims=True).astype(jnp.int32)

    fx0 = jnp.zeros((GB, 1), jnp.int32)
    zacc = (jnp.zeros((GB, NCOL), jnp.int32),
            jnp.zeros((GB, NCOL), jnp.float32),
            jnp.zeros((GB, NCOL), jnp.float32),
            jnp.zeros((GB, NCOL), jnp.float32))
    m8a0, rmin8a0, acc_a, st0 = s1(0, 0, fx0, zacc)
    st0()
    fx_a = s3(s2(m8a0, rmin8a0))

    def body(k, carry):
        fx_a, acc_a = carry
        m8a, rmin8a, acc_a, store_acc = s1(0, k + 1, fx_a, acc_a)
        store_acc()
        cand_a = s2(m8a, rmin8a)
        new_fx_a = s3(cand_a)
        return new_fx_a, acc_a

    lax.fori_loop(0, S - 1, body, (fx_a, acc_a), unroll=False)


def _fps_pallas(x0, x1, x2, interpret=False):
    out_shape = [
        jax.ShapeDtypeStruct((B, SROW, NCOL), jnp.int32),
        jax.ShapeDtypeStruct((B, SROW, NCOL), jnp.float32),
        jax.ShapeDtypeStruct((B, SROW, NCOL), jnp.float32),
        jax.ShapeDtypeStruct((B, SROW, NCOL), jnp.float32),
    ]
    scratch = []
    for g in range(G):
        scratch.append(pltpu.VMEM((GB, NROW, NCOL), jnp.float32))
    return pl.pallas_call(
        _fps_body,
        out_shape=out_shape,
        scratch_shapes=scratch,
        interpret=interpret,
    )(x0, x1, x2)


def _mm_body(g_ref, w_ref, bias_ref, out_ref):
    for b in range(B):
        out_ref[b] = lax.dot_general(
            w_ref[...], g_ref[b], (((1,), (1,)), ((), ())),
            preferred_element_type=jnp.float32,
            precision=lax.Precision.HIGHEST) + bias_ref[...]


def _mm_pallas(g, w, bias, interpret=False):
    return pl.pallas_call(
        _mm_body,
        out_shape=jax.ShapeDtypeStruct((B, COUT, S), jnp.float32),
        interpret=interpret,
    )(g, w, bias)


_BPW = (B * S) // 32
_IDX_ROWS = _BPW // NCOL


def _sc_gather_body(table_ref, idx_ref, out_ref, idx_v, rows_v, sem):
    wid = lax.axis_index("s") * 2 + lax.axis_index("c")
    pltpu.sync_copy(idx_ref.at[pl.ds(wid * _IDX_ROWS, _IDX_ROWS)], idx_v)
    for j in range(_IDX_ROWS):
        pltpu.async_copy(table_ref.at[idx_v.at[j]],
                         rows_v.at[pl.ds(j * NCOL, NCOL)], sem).wait()
    pltpu.sync_copy(rows_v, out_ref.at[pl.ds(wid * _BPW, _BPW)])


def _sc_gather(table, idx2d):
    mesh = plsc.VectorSubcoreMesh(core_axis_name="c", subcore_axis_name="s")
    kern = pl.kernel(
        _sc_gather_body,
        mesh=mesh,
        out_type=jax.ShapeDtypeStruct((B * S, CIN), jnp.float32),
        scratch_types=[
            pltpu.VMEM((_IDX_ROWS, NCOL), jnp.int32),
            pltpu.VMEM((_BPW, CIN), jnp.float32),
            pltpu.SemaphoreType.DMA,
        ],
    )
    return kern(table, idx2d)


def kernel(xyz, x, W, b):
    x0 = xyz[:, :, 0].reshape(B, NROW, NCOL)
    x1 = xyz[:, :, 1].reshape(B, NROW, NCOL)
    x2 = xyz[:, :, 2].reshape(B, NROW, NCOL)
    idx, n0, n1, n2 = _fps_pallas(x0, x1, x2)
    new_xyz = jnp.stack([n0.reshape(B, S), n1.reshape(B, S),
                         n2.reshape(B, S)], axis=-1)
    table = jnp.transpose(x, (0, 2, 1)).reshape(B * N, CIN)
    g = _sc_gather(table, idx.reshape((B * S) // NCOL, NCOL))
    new_x = _mm_pallas(g.reshape(B, S, CIN), W, b.reshape(COUT, 1))
    return (new_xyz, new_x)
